# baseline (speedup 1.0000x reference)
import functools
import jax
import jax.numpy as jnp
from jax import lax
from jax.experimental import pallas as pl
from jax.experimental.pallas import tpu as pltpu
from jax.experimental.pallas import tpu_sc as plsc

K = 20
NEG = float('-inf')
IBIG = 1 << 30
RBLK = 256

NC, NS = 2, 16
NW = NC * NS


def _pq_body(xT_ref, Wc_ref, bc_ref, pq_ref):
    xt = xT_ref[0]
    w = Wc_ref[...]
    pq_ref[0] = jnp.dot(xt, w, preferred_element_type=jnp.float32) + bc_ref[...]


def _topk_body(xb_ref, xtr_ref, idx_ref, xx_ref):
    b = pl.program_id(0)
    i = pl.program_id(1)
    xb = xb_ref[0]
    n = xb.shape[1]

    @pl.when(i == 0)
    def _():
        xx_ref[...] = jnp.sum(xb * xb, axis=0, keepdims=True)

    xtr = xtr_ref[0]
    s = 2.0 * jnp.dot(xtr, xb, preferred_element_type=jnp.float32) - xx_ref[...]
    base = b * n
    gidx = jax.lax.broadcasted_iota(jnp.int32, (RBLK, n), 1) + base
    rowg = jax.lax.broadcasted_iota(jnp.int32, (RBLK, 1), 0) + (i * RBLK + base)
    s = jnp.where(gidx == rowg, NEG, s)
    lane = jax.lax.broadcasted_iota(jnp.int32, (RBLK, K), 1)
    acc = jnp.where(lane == 0, rowg, 0)
    for t in range(1, K):
        m = jnp.max(s, axis=1, keepdims=True)
        j = jnp.min(jnp.where(s == m, gidx, IBIG), axis=1, keepdims=True)
        acc = jnp.where(lane == t, j, acc)
        if t < K - 1:
            s = jnp.where(gidx == j, NEG, s)
    idx_ref[0] = acc


def _make_gather_max(BN, O, IPW, G, NIR):
    NGRP = BN // NW // G
    mesh = plsc.VectorSubcoreMesh(core_axis_name="c", subcore_axis_name="s")

    @functools.partial(
        pl.kernel,
        out_type=jax.ShapeDtypeStruct((BN, O), jnp.float32),
        mesh=mesh,
        scratch_types=[
            pltpu.VMEM((IPW, 128), jnp.int32),
            pltpu.VMEM((G * K, 2 * O), jnp.float32),
            pltpu.VMEM((G, O), jnp.float32),
            pltpu.SemaphoreType.DMA,
        ],
    )
    def gather_max(pq_hbm, idx_hbm, out_hbm, idx_v, rows_v, o_v, gsem):
        wid = lax.axis_index("s") * NC + lax.axis_index("c")
        pltpu.sync_copy(idx_hbm.at[pl.ds(wid * IPW, IPW)], idx_v)

        def group(g, carry):
            base_pt = wid * (BN // NW) + g * G
            for j in range(NIR):
                pltpu.async_copy(
                    pq_hbm.at[idx_v.at[g * NIR + j]],
                    rows_v.at[pl.ds(j * 128, 128)],
                    gsem,
                ).wait()

            def point(p, c2):
                r0 = p * K
                for c in range(O // 16):
                    sl = pl.ds(c * 16, 16)
                    acc = rows_v[r0, sl]
                    for t in range(1, K):
                        acc = jnp.maximum(acc, rows_v[r0 + t, sl])
                    z = acc + rows_v[r0, pl.ds(O + c * 16, 16)]
                    o_v[p, sl] = jnp.where(z >= 0, z, 0.2 * z)
                return c2

            lax.fori_loop(0, G, point, 0)
            pltpu.sync_copy(o_v, out_hbm.at[pl.ds(base_pt, G)])
            return carry

        lax.fori_loop(0, NGRP, group, 0)

    return gather_max


def kernel(x, W, b):
    B, C, N = x.shape
    O = W.shape[0]
    BN = B * N
    W1 = W[:, :C]
    W2 = W[:, C:]
    Wcat = jnp.concatenate([W1.T, (W2 - W1).T], axis=1)
    bcat = jnp.concatenate([jnp.zeros((O,), W.dtype), b])[None, :]
    xT = jnp.transpose(x, (0, 2, 1))

    pq = pl.pallas_call(
        _pq_body,
        grid=(B,),
        in_specs=[
            pl.BlockSpec((1, N, C), lambda i: (i, 0, 0)),
            pl.BlockSpec((C, 2 * O), lambda i: (0, 0)),
            pl.BlockSpec((1, 2 * O), lambda i: (0, 0)),
        ],
        out_specs=pl.BlockSpec((1, N, 2 * O), lambda i: (i, 0, 0)),
        out_shape=jax.ShapeDtypeStruct((B, N, 2 * O), jnp.float32),
    )(xT, Wcat, bcat)

    idx = pl.pallas_call(
        _topk_body,
        grid=(B, N // RBLK),
        in_specs=[
            pl.BlockSpec((1, C, N), lambda b_, i: (b_, 0, 0)),
            pl.BlockSpec((1, RBLK, C), lambda b_, i: (b_, i, 0)),
        ],
        out_specs=pl.BlockSpec((1, RBLK, K), lambda b_, i: (b_, i, 0)),
        out_shape=jax.ShapeDtypeStruct((B, N, K), jnp.int32),
        scratch_shapes=[pltpu.VMEM((1, N), jnp.float32)],
    )(x, xT)

    G = 32
    NIR = G * K // 128
    IPW = BN // NW * K // 128
    idx_flat = idx.reshape(BN * K // 128, 128)
    pq_flat = pq.reshape(BN, 2 * O)
    outT = _make_gather_max(BN, O, IPW, G, NIR)(pq_flat, idx_flat)
    return jnp.transpose(outT.reshape(B, N, O), (0, 2, 1))


# speedup vs baseline: 1.2506x; 1.2506x over previous
import functools
import jax
import jax.numpy as jnp
from jax import lax
from jax.experimental import pallas as pl
from jax.experimental.pallas import tpu as pltpu
from jax.experimental.pallas import tpu_sc as plsc

K = 20
NEG = float('-inf')
FK = float(20)
OFFF = float(1 << 19)
BIGF = 3.0e38
RBLK = 256

NC, NS = 2, 16
NW = NC * NS


def _pq_body(xT_ref, Wc_ref, bc_ref, pq_ref):
    xt = xT_ref[0]
    w = Wc_ref[...]
    pq_ref[0] = jnp.dot(xt, w, preferred_element_type=jnp.float32) + bc_ref[...]


def _topk_body(xb_ref, xtr_ref, idx_ref, xx_ref):
    b = pl.program_id(0)
    i = pl.program_id(1)
    xb = xb_ref[0]
    n = xb.shape[1]

    @pl.when(i == 0)
    def _():
        xx_ref[...] = jnp.sum(xb * xb, axis=0, keepdims=True)

    xtr = xtr_ref[0]
    s = 2.0 * jnp.dot(xtr, xb, preferred_element_type=jnp.float32) - xx_ref[...]

    jotaf = jax.lax.broadcasted_iota(jnp.int32, (RBLK, n), 1).astype(jnp.float32)
    rowf = (jax.lax.broadcasted_iota(jnp.int32, (RBLK, 1), 0)
            + i * RBLK).astype(jnp.float32)
    s = jnp.where(jotaf == rowf, NEG, s)
    lane = jax.lax.broadcasted_iota(jnp.int32, (RBLK, K), 1)
    accf = jnp.where(lane == 0, rowf, 0.0)
    for t in range(1, K):
        m = jnp.max(s, axis=1, keepdims=True)
        jf = jnp.min(jnp.where(s == m, jotaf, BIGF), axis=1, keepdims=True)
        accf = jnp.where(lane == t, jf, accf)
        if t < K - 1:
            s = jnp.where(jotaf == jf, NEG, s)
    idx_ref[0] = accf.astype(jnp.int32) + b * n


def _make_gather_max(BN, O, IPW, G, NIR):
    NGRP = BN // NW // G
    mesh = plsc.VectorSubcoreMesh(core_axis_name="c", subcore_axis_name="s")

    @functools.partial(
        pl.kernel,
        out_type=jax.ShapeDtypeStruct((BN, O), jnp.float32),
        mesh=mesh,
        scratch_types=[
            pltpu.VMEM((IPW, 128), jnp.int32),
            pltpu.VMEM((G * K, 2 * O), jnp.float32),
            pltpu.VMEM((G, O), jnp.float32),
            pltpu.SemaphoreType.DMA,
        ],
    )
    def gather_max(pq_hbm, idx_hbm, out_hbm, idx_v, rows_v, o_v, gsem):
        wid = lax.axis_index("s") * NC + lax.axis_index("c")
        pltpu.sync_copy(idx_hbm.at[pl.ds(wid * IPW, IPW)], idx_v)

        def group(g, carry):
            base_pt = wid * (BN // NW) + g * G
            for j in range(NIR):
                pltpu.async_copy(
                    pq_hbm.at[idx_v.at[g * NIR + j]],
                    rows_v.at[pl.ds(j * 128, 128)],
                    gsem,
                ).wait()

            def point(p, c2):
                r0 = p * K
                for c in range(O // 16):
                    sl = pl.ds(c * 16, 16)
                    acc = rows_v[r0, sl]
                    for t in range(1, K):
                        acc = jnp.maximum(acc, rows_v[r0 + t, sl])
                    o_v[p, sl] = acc
                return c2

            lax.fori_loop(0, G, point, 0)
            pltpu.sync_copy(o_v, out_hbm.at[pl.ds(base_pt, G)])
            return carry

        lax.fori_loop(0, NGRP, group, 0)

    return gather_max


def _fin_body(m_ref, pq_ref, eye_ref, o_ref):
    o = m_ref.shape[2]
    z = m_ref[0] + pq_ref[0][:, o:]
    z = jnp.where(z >= 0, z, 0.2 * z)
    o_ref[0] = jax.lax.dot_general(
        eye_ref[...], z, (((0,), (1,)), ((), ())),
        preferred_element_type=jnp.float32)


def kernel(x, W, b):
    B, C, N = x.shape
    O = W.shape[0]
    BN = B * N
    W1 = W[:, :C]
    W2 = W[:, C:]
    Wcat = jnp.concatenate([W1.T, (W2 - W1).T], axis=1)
    bcat = jnp.concatenate([jnp.zeros((O,), W.dtype), b])[None, :]
    xT = jnp.transpose(x, (0, 2, 1))

    pq = pl.pallas_call(
        _pq_body,
        grid=(B,),
        in_specs=[
            pl.BlockSpec((1, N, C), lambda i: (i, 0, 0)),
            pl.BlockSpec((C, 2 * O), lambda i: (0, 0)),
            pl.BlockSpec((1, 2 * O), lambda i: (0, 0)),
        ],
        out_specs=pl.BlockSpec((1, N, 2 * O), lambda i: (i, 0, 0)),
        out_shape=jax.ShapeDtypeStruct((B, N, 2 * O), jnp.float32),
    )(xT, Wcat, bcat)

    idx = pl.pallas_call(
        _topk_body,
        grid=(B, N // RBLK),
        in_specs=[
            pl.BlockSpec((1, C, N), lambda b_, i: (b_, 0, 0)),
            pl.BlockSpec((1, RBLK, C), lambda b_, i: (b_, i, 0)),
        ],
        out_specs=pl.BlockSpec((1, RBLK, K), lambda b_, i: (b_, i, 0)),
        out_shape=jax.ShapeDtypeStruct((B, N, K), jnp.int32),
        scratch_shapes=[pltpu.VMEM((1, N), jnp.float32)],
    )(x, xT)

    G = 32
    NIR = G * K // 128
    IPW = BN // NW * K // 128
    idx_flat = idx.reshape(BN * K // 128, 128)
    pq_flat = pq.reshape(BN, 2 * O)
    M = _make_gather_max(BN, O, IPW, G, NIR)(pq_flat, idx_flat)

    FB = 512
    eye = jnp.eye(O, dtype=jnp.float32)
    return pl.pallas_call(
        _fin_body,
        grid=(B, N // FB),
        in_specs=[
            pl.BlockSpec((1, FB, O), lambda b_, i: (b_, i, 0)),
            pl.BlockSpec((1, FB, 2 * O), lambda b_, i: (b_, i, 0)),
            pl.BlockSpec((O, O), lambda b_, i: (0, 0)),
        ],
        out_specs=pl.BlockSpec((1, O, FB), lambda b_, i: (b_, 0, i)),
        out_shape=jax.ShapeDtypeStruct((B, O, N), jnp.float32),
    )(M.reshape(B, N, O), pq, eye)


# speedup vs baseline: 1.3953x; 1.1157x over previous
import functools
import jax
import jax.numpy as jnp
from jax import lax
from jax.experimental import pallas as pl
from jax.experimental.pallas import tpu as pltpu
from jax.experimental.pallas import tpu_sc as plsc

K = 20
NEG = float('-inf')
BIGF = 3.0e38
RBLK = 256
IDXW = 64

NC, NS = 2, 16
NW = NC * NS


def _pq_body(xT_ref, Wc_ref, bc_ref, pq_ref):
    xt = xT_ref[0]
    w = Wc_ref[...]
    pq_ref[0] = jnp.dot(xt, w, preferred_element_type=jnp.float32) + bc_ref[...]


def _topk_body(xb_ref, xtr_ref, idx_ref, xx_ref, *, boff):
    b = pl.program_id(0)
    i = pl.program_id(1)
    xb = xb_ref[0]
    n = xb.shape[1]

    @pl.when(i == 0)
    def _():
        xx_ref[...] = jnp.sum(xb * xb, axis=0, keepdims=True)

    xtr = xtr_ref[0]
    s = 2.0 * jnp.dot(xtr, xb, preferred_element_type=jnp.float32) - xx_ref[...]

    jotaf = jax.lax.broadcasted_iota(jnp.int32, (RBLK, n), 1).astype(jnp.float32)
    rowf = (jax.lax.broadcasted_iota(jnp.int32, (RBLK, 1), 0)
            + i * RBLK).astype(jnp.float32)
    s = jnp.where(jotaf == rowf, NEG, s)
    lane = jax.lax.broadcasted_iota(jnp.int32, (RBLK, K), 1)
    accf = jnp.where(lane == 0, rowf, 0.0)
    for t in range(1, K):
        m = jnp.max(s, axis=1, keepdims=True)
        jf = jnp.min(jnp.where(s == m, jotaf, BIGF), axis=1, keepdims=True)
        accf = jnp.where(lane == t, jf, accf)
        if t < K - 1:
            s = jnp.where(jotaf == jf, NEG, s)
    idx_ref[0] = accf.astype(jnp.int32) + (b + boff) * n


def _make_gather_max(BNH, O, G):
    PW = BNH // NW
    NGRP = PW // G
    NIR = G * K // IDXW
    IPW = PW * K // IDXW
    mesh = plsc.VectorSubcoreMesh(core_axis_name="c", subcore_axis_name="s")

    @functools.partial(
        pl.kernel,
        out_type=jax.ShapeDtypeStruct((BNH, O), jnp.float32),
        mesh=mesh,
        scratch_types=[
            pltpu.VMEM((IPW, IDXW), jnp.int32),
            pltpu.VMEM((G * K, 2 * O), jnp.float32),
            pltpu.VMEM((G * K, 2 * O), jnp.float32),
            pltpu.VMEM((G, O), jnp.float32),
            pltpu.SemaphoreType.DMA,
            pltpu.SemaphoreType.DMA,
        ],
    )
    def gather_max(pq_hbm, idx_hbm, out_hbm, idx_v, rows0, rows1, o_v, sem0, sem1):
        wid = lax.axis_index("s") * NC + lax.axis_index("c")
        pltpu.sync_copy(idx_hbm.at[pl.ds(wid * IPW, IPW)], idx_v)

        def fire(g, buf, sem):
            for j in range(NIR):
                pltpu.async_copy(
                    pq_hbm.at[idx_v.at[g * NIR + j]],
                    buf.at[pl.ds(j * IDXW, IDXW)],
                    sem)

        def drain(buf, sem):
            pltpu.make_async_copy(pq_hbm.at[pl.ds(0, G * K)], buf, sem).wait()

        def compute(g, buf):
            def point(p, c2):
                r0 = p * K
                for c in range(O // 16):
                    sl = pl.ds(c * 16, 16)
                    acc = buf[r0, sl]
                    for t in range(1, K):
                        acc = jnp.maximum(acc, buf[r0 + t, sl])
                    o_v[p, sl] = acc
                return c2

            lax.fori_loop(0, G, point, 0)
            pltpu.sync_copy(o_v, out_hbm.at[pl.ds(wid * PW + g * G, G)])

        fire(0, rows0, sem0)

        @pl.loop(0, NGRP // 2)
        def _(go):
            g0 = go * 2
            fire(g0 + 1, rows1, sem1)
            drain(rows0, sem0)
            compute(g0, rows0)

            @pl.when(g0 + 2 < NGRP)
            def _():
                fire(g0 + 2, rows0, sem0)

            drain(rows1, sem1)
            compute(g0 + 1, rows1)

    return gather_max


def _fin_body(m_ref, pq_ref, eye_ref, o_ref):
    o = m_ref.shape[2]
    z = m_ref[0] + pq_ref[0][:, o:]
    z = jnp.where(z >= 0, z, 0.2 * z)
    o_ref[0] = jax.lax.dot_general(
        eye_ref[...], z, (((0,), (1,)), ((), ())),
        preferred_element_type=jnp.float32)


def kernel(x, W, b):
    B, C, N = x.shape
    O = W.shape[0]
    BN = B * N
    W1 = W[:, :C]
    W2 = W[:, C:]
    Wcat = jnp.concatenate([W1.T, (W2 - W1).T], axis=1)
    bcat = jnp.concatenate([jnp.zeros((O,), W.dtype), b])[None, :]
    xT = jnp.transpose(x, (0, 2, 1))

    pq = pl.pallas_call(
        _pq_body,
        grid=(B,),
        in_specs=[
            pl.BlockSpec((1, N, C), lambda i: (i, 0, 0)),
            pl.BlockSpec((C, 2 * O), lambda i: (0, 0)),
            pl.BlockSpec((1, 2 * O), lambda i: (0, 0)),
        ],
        out_specs=pl.BlockSpec((1, N, 2 * O), lambda i: (i, 0, 0)),
        out_shape=jax.ShapeDtypeStruct((B, N, 2 * O), jnp.float32),
    )(xT, Wcat, bcat)
    pq_flat = pq.reshape(BN, 2 * O)

    BH = B // 2
    BNH = BH * N
    G = 16
    FB = 512
    eye = jnp.eye(O, dtype=jnp.float32)
    sc_call = _make_gather_max(BNH, O, G)
    halves = []
    for h in range(2):
        xh = x[h * BH:(h + 1) * BH]
        xTh = xT[h * BH:(h + 1) * BH]
        idx = pl.pallas_call(
            functools.partial(_topk_body, boff=h * BH),
            grid=(BH, N // RBLK),
            in_specs=[
                pl.BlockSpec((1, C, N), lambda b_, i: (b_, 0, 0)),
                pl.BlockSpec((1, RBLK, C), lambda b_, i: (b_, i, 0)),
            ],
            out_specs=pl.BlockSpec((1, RBLK, K), lambda b_, i: (b_, i, 0)),
            out_shape=jax.ShapeDtypeStruct((BH, N, K), jnp.int32),
            scratch_shapes=[pltpu.VMEM((1, N), jnp.float32)],
        )(xh, xTh)
        idx_flat = idx.reshape(BNH * K // IDXW, IDXW)
        M = sc_call(pq_flat, idx_flat)
        outh = pl.pallas_call(
            _fin_body,
            grid=(BH, N // FB),
            in_specs=[
                pl.BlockSpec((1, FB, O), lambda b_, i: (b_, i, 0)),
                pl.BlockSpec((1, FB, 2 * O), lambda b_, i: (b_, i, 0)),
                pl.BlockSpec((O, O), lambda b_, i: (0, 0)),
            ],
            out_specs=pl.BlockSpec((1, O, FB), lambda b_, i: (b_, 0, i)),
            out_shape=jax.ShapeDtypeStruct((BH, O, N), jnp.float32),
        )(M.reshape(BH, N, O), pq[h * BH:(h + 1) * BH], eye)
        halves.append(outh)
    return jnp.concatenate(halves, axis=0)


# speedup vs baseline: 1.4272x; 1.0229x over previous
import functools
import jax
import jax.numpy as jnp
from jax import lax
from jax.experimental import pallas as pl
from jax.experimental.pallas import tpu as pltpu
from jax.experimental.pallas import tpu_sc as plsc

K = 20
NEG = float('-inf')
BIGF = 3.0e38
RBLK = 256
IDXW = 64

NC, NS = 2, 16
NW = NC * NS


def _pq_body(xT_ref, Wc_ref, bc_ref, pq_ref):
    xt = xT_ref[0]
    w = Wc_ref[...]
    pq_ref[0] = jnp.dot(xt, w, preferred_element_type=jnp.float32) + bc_ref[...]


def _topk_body(xb_ref, xtr_ref, idx_ref, xx_ref, *, boff):
    b = pl.program_id(0)
    i = pl.program_id(1)
    xb = xb_ref[0]
    n = xb.shape[1]

    @pl.when(i == 0)
    def _():
        xx_ref[...] = jnp.sum(xb * xb, axis=0, keepdims=True)

    xtr = xtr_ref[0]
    s = 2.0 * jnp.dot(xtr, xb, preferred_element_type=jnp.float32) - xx_ref[...]

    jotaf = jax.lax.broadcasted_iota(jnp.int32, (RBLK, n), 1).astype(jnp.float32)
    rowf = (jax.lax.broadcasted_iota(jnp.int32, (RBLK, 1), 0)
            + i * RBLK).astype(jnp.float32)
    s = jnp.where(jotaf == rowf, NEG, s)
    lane = jax.lax.broadcasted_iota(jnp.int32, (RBLK, K), 1)
    accf = jnp.where(lane == 0, rowf, 0.0)
    for t in range(1, K):
        m = jnp.max(s, axis=1, keepdims=True)
        jf = jnp.min(jnp.where(s == m, jotaf, BIGF), axis=1, keepdims=True)
        accf = jnp.where(lane == t, jf, accf)
        if t < K - 1:
            s = jnp.where(jotaf == jf, NEG, s)
    idx_ref[0] = accf.astype(jnp.int32) + (b + boff) * n


def _make_gather_max(BNH, O, G):
    PW = BNH // NW
    NGRP = PW // G
    NIR = G * K // IDXW
    IPW = PW * K // IDXW
    mesh = plsc.VectorSubcoreMesh(core_axis_name="c", subcore_axis_name="s")

    @functools.partial(
        pl.kernel,
        out_type=jax.ShapeDtypeStruct((BNH, O), jnp.float32),
        mesh=mesh,
        scratch_types=[
            pltpu.VMEM((IPW, IDXW), jnp.int32),
            pltpu.VMEM((G * K, 2 * O), jnp.float32),
            pltpu.VMEM((G * K, 2 * O), jnp.float32),
            pltpu.VMEM((G, O), jnp.float32),
            pltpu.SemaphoreType.DMA,
            pltpu.SemaphoreType.DMA,
        ],
    )
    def gather_max(pq_hbm, idx_hbm, out_hbm, idx_v, rows0, rows1, o_v, sem0, sem1):
        wid = lax.axis_index("s") * NC + lax.axis_index("c")
        pltpu.sync_copy(idx_hbm.at[pl.ds(wid * IPW, IPW)], idx_v)

        def fire(g, buf, sem):
            for j in range(NIR):
                pltpu.async_copy(
                    pq_hbm.at[idx_v.at[g * NIR + j]],
                    buf.at[pl.ds(j * IDXW, IDXW)],
                    sem)

        def drain(buf, sem):
            pltpu.make_async_copy(pq_hbm.at[pl.ds(0, G * K)], buf, sem).wait()

        def compute(g, buf):
            def point(p, c2):
                r0 = p * K
                for c in range(O // 16):
                    sl = pl.ds(c * 16, 16)
                    acc = buf[r0, sl]
                    for t in range(1, K):
                        acc = jnp.maximum(acc, buf[r0 + t, sl])
                    o_v[p, sl] = acc
                return c2

            lax.fori_loop(0, G, point, 0)
            pltpu.sync_copy(o_v, out_hbm.at[pl.ds(wid * PW + g * G, G)])

        fire(0, rows0, sem0)

        @pl.loop(0, NGRP // 2)
        def _(go):
            g0 = go * 2
            fire(g0 + 1, rows1, sem1)
            drain(rows0, sem0)
            compute(g0, rows0)

            @pl.when(g0 + 2 < NGRP)
            def _():
                fire(g0 + 2, rows0, sem0)

            drain(rows1, sem1)
            compute(g0 + 1, rows1)

    return gather_max


def _fin_body(m_ref, pq_ref, eye_ref, o_ref):
    o = m_ref.shape[2]
    z = m_ref[0] + pq_ref[0][:, o:]
    z = jnp.where(z >= 0, z, 0.2 * z)
    o_ref[0] = jax.lax.dot_general(
        eye_ref[...], z, (((0,), (1,)), ((), ())),
        preferred_element_type=jnp.float32)


def kernel(x, W, b):
    B, C, N = x.shape
    O = W.shape[0]
    BN = B * N
    W1 = W[:, :C]
    W2 = W[:, C:]
    Wcat = jnp.concatenate([W1.T, (W2 - W1).T], axis=1)
    bcat = jnp.concatenate([jnp.zeros((O,), W.dtype), b])[None, :]
    xT = jnp.transpose(x, (0, 2, 1))

    pq = pl.pallas_call(
        _pq_body,
        grid=(B,),
        in_specs=[
            pl.BlockSpec((1, N, C), lambda i: (i, 0, 0)),
            pl.BlockSpec((C, 2 * O), lambda i: (0, 0)),
            pl.BlockSpec((1, 2 * O), lambda i: (0, 0)),
        ],
        out_specs=pl.BlockSpec((1, N, 2 * O), lambda i: (i, 0, 0)),
        out_shape=jax.ShapeDtypeStruct((B, N, 2 * O), jnp.float32),
    )(xT, Wcat, bcat)
    pq_flat = pq.reshape(BN, 2 * O)

    BH = B // 4
    BNH = BH * N
    G = 16
    FB = 512
    eye = jnp.eye(O, dtype=jnp.float32)
    sc_call = _make_gather_max(BNH, O, G)
    halves = []
    for h in range(4):
        xh = x[h * BH:(h + 1) * BH]
        xTh = xT[h * BH:(h + 1) * BH]
        idx = pl.pallas_call(
            functools.partial(_topk_body, boff=h * BH),
            grid=(BH, N // RBLK),
            in_specs=[
                pl.BlockSpec((1, C, N), lambda b_, i: (b_, 0, 0)),
                pl.BlockSpec((1, RBLK, C), lambda b_, i: (b_, i, 0)),
            ],
            out_specs=pl.BlockSpec((1, RBLK, K), lambda b_, i: (b_, i, 0)),
            out_shape=jax.ShapeDtypeStruct((BH, N, K), jnp.int32),
            scratch_shapes=[pltpu.VMEM((1, N), jnp.float32)],
        )(xh, xTh)
        idx_flat = idx.reshape(BNH * K // IDXW, IDXW)
        M = sc_call(pq_flat, idx_flat)
        outh = pl.pallas_call(
            _fin_body,
            grid=(BH, N // FB),
            in_specs=[
                pl.BlockSpec((1, FB, O), lambda b_, i: (b_, i, 0)),
                pl.BlockSpec((1, FB, 2 * O), lambda b_, i: (b_, i, 0)),
                pl.BlockSpec((O, O), lambda b_, i: (0, 0)),
            ],
            out_specs=pl.BlockSpec((1, O, FB), lambda b_, i: (b_, 0, i)),
            out_shape=jax.ShapeDtypeStruct((BH, O, N), jnp.float32),
        )(M.reshape(BH, N, O), pq[h * BH:(h + 1) * BH], eye)
        halves.append(outh)
    return jnp.concatenate(halves, axis=0)
